# baseline (speedup 1.0000x reference)
import functools
import math

import jax
import jax.numpy as jnp
from jax import lax
from jax.experimental import pallas as pl
from jax.experimental.pallas import tpu as pltpu
from jax.experimental.pallas import tpu_sc as plsc

B, N, KNN = 4, 4096, 9
NG = 8
C = 10
QB = 1024
NTILES = 32
QPT = (B * N) // NTILES
BIGF = 3.0e38


def _knn_body(q_ref, d_ref, out_ref):
    q = q_ref[0]
    d = d_ref[0]
    qx, qy, qz = q[:, 0:1], q[:, 1:2], q[:, 2:3]
    dx, dy, dz = d[0:1, :], d[1:2, :], d[2:3, :]
    bq = lambda v: v.astype(jnp.bfloat16).astype(jnp.float32)
    t = bq(qx) * bq(dx) + bq(qy) * bq(dy) + bq(qz) * bq(dz)
    qn = qx * qx + qy * qy + qz * qz
    dn = dx * dx + dy * dy + dz * dz
    dist = -2.0 * t + qn + dn
    iota = lax.broadcasted_iota(jnp.int32, (QB, N), 1).astype(jnp.float32)
    lane16 = lax.broadcasted_iota(jnp.int32, (QB, 16), 1)
    outv = jnp.zeros((QB, 16), jnp.int32)
    for j in range(KNN):
        m = jnp.min(dist, axis=1, keepdims=True)
        cand = jnp.where(dist == m, iota, jnp.float32(N))
        idx = jnp.min(cand, axis=1, keepdims=True)
        outv = jnp.where(lane16 == j, idx.astype(jnp.int32), outv)
        if j + 1 < KNN:
            dist = jnp.where(iota == idx, BIGF, dist)
    out_ref[0] = outv


def _knn_call(center, center_t):
    return pl.pallas_call(
        _knn_body,
        grid=(B, N // QB),
        in_specs=[
            pl.BlockSpec((1, QB, 3), lambda b, q: (b, q, 0)),
            pl.BlockSpec((1, 3, N), lambda b, q: (b, 0, 0)),
        ],
        out_specs=pl.BlockSpec((1, QB, 16), lambda b, q: (b, q, 0)),
        out_shape=jax.ShapeDtypeStruct((B, N, 16), jnp.int32),
    )(center, center_t)


def _sc_gather_body(tab_hbm, idx_hbm, out_hbm, tab_v, idx_v, out_v):
    wid = lax.axis_index("s") * 2 + lax.axis_index("c")
    b = wid // (N // QPT)
    pltpu.sync_copy(tab_hbm.at[b], tab_v)
    pltpu.sync_copy(idx_hbm.at[wid], idx_v)

    def chunk(i, _):
        for j in range(NG):
            iv = idx_v[j, pl.ds(i * 16, 16)]
            for c in range(3):
                out_v[c, j, pl.ds(i * 16, 16)] = plsc.load_gather(
                    tab_v, [iv + c])
        return _

    lax.fori_loop(0, QPT // 16, chunk, 0)
    pltpu.sync_copy(out_v, out_hbm.at[wid])


def _sc_gather_call(tab, gidx3):
    kfn = functools.partial(
        pl.kernel,
        mesh=plsc.VectorSubcoreMesh(core_axis_name="c", subcore_axis_name="s"),
        compiler_params=pltpu.CompilerParams(needs_layout_passes=False),
        out_type=jax.ShapeDtypeStruct((NTILES, 3, NG, QPT), jnp.float32),
        scratch_types=[
            pltpu.VMEM((3 * N,), jnp.float32),
            pltpu.VMEM((NG, QPT), jnp.int32),
            pltpu.VMEM((3, NG, QPT), jnp.float32),
        ],
    )(_sc_gather_body)
    return kfn(tab, gidx3)


_EPS = 1e-10
_CE_PAIRS = [(0, 1), (2, 3), (4, 5), (6, 7),
             (0, 2), (1, 3), (4, 6), (5, 7),
             (1, 2), (5, 6),
             (0, 4), (1, 5), (2, 6), (3, 7),
             (2, 4), (3, 5),
             (1, 2), (3, 4), (5, 6)]


def _atan2_phi(y, x):
    xa = x + 1e-10 * (jnp.abs(x) < 1e-10).astype(jnp.float32)
    ya = y + 1e-10 * (jnp.abs(y) < 1e-10).astype(jnp.float32)
    return jnp.arctan2(ya, xa) / (2.0 * math.pi) + 0.5


def _geom_body(xg_ref, c_ref, feat_ref):
    cx, cy, cz = c_ref[0, 0], c_ref[0, 1], c_ref[0, 2]
    phis, js, xs, ys, zs = [], [], [], [], []
    for j in range(NG):
        x = xg_ref[0, j, 0] - cx
        y = xg_ref[1, j, 0] - cy
        z = xg_ref[2, j, 0] - cz
        xs.append(x); ys.append(y); zs.append(z)
        phis.append(_atan2_phi(y, x))
        js.append(jnp.full(x.shape, j, jnp.int32))
    for (a, b) in _CE_PAIRS:
        swap = (phis[a] > phis[b]) | ((phis[a] == phis[b]) & (js[a] > js[b]))
        for arr in (phis, js, xs, ys, zs):
            na = jnp.where(swap, arr[b], arr[a])
            nb = jnp.where(swap, arr[a], arr[b])
            arr[a], arr[b] = na, nb

    sq3 = jnp.float32(math.sqrt(3.0) + 1e-6)
    smask = None
    feats = []
    for i in range(NG):
        k = (i + 1) % NG
        sx, sy, sz = xs[i], ys[i], zs[i]
        rx, ry, rz = xs[k], ys[k], zs[k]
        nx = sy * rz - sz * ry
        ny = sz * rx - sx * rz
        nz = sx * ry - sy * rx
        norm = jnp.sqrt(nx * nx + ny * ny + nz * nz)
        safe = jnp.where(norm < 1e-6, jnp.float32(1.0), norm)
        ux, uy, uz = nx / safe, ny / safe, nz / safe
        if i == 0:
            smask = jnp.where(ux > 0, jnp.float32(1.0), jnp.float32(-1.0))
        feats.append((ux, uy, uz, sx, sy, sz, rx, ry, rz))

    for i in range(NG):
        ux, uy, uz, sx, sy, sz, rx, ry, rz = feats[i]
        ux, uy, uz = ux * smask, uy * smask, uz * smask
        gx, gy, gz = (sx + rx) / 3.0, (sy + ry) / 3.0, (sz + rz) / 3.0
        rho = jnp.sqrt(gx * gx + gy * gy + gz * gz + _EPS)
        zdr = gz / jnp.maximum(rho, _EPS)
        zdr = jnp.clip(zdr, -1.0 + _EPS, 1.0 - _EPS)
        theta = jnp.arctan2(jnp.sqrt((1.0 - zdr) * (1.0 + zdr)), zdr) / math.pi
        phi = _atan2_phi(gy, gx)
        nn = jnp.sqrt(ux * ux + uy * uy + uz * uz)
        nsafe = jnp.maximum(nn, 1e-6)
        small = nn < 1e-6
        snx = jnp.where(small, jnp.float32(1.0), ux / nsafe)
        sny = jnp.where(small, jnp.float32(0.0), uy / nsafe)
        snz = jnp.where(small, jnp.float32(0.0), uz / nsafe)
        ccx = jnp.clip(gx, -1e6, 1e6)
        ccy = jnp.clip(gy, -1e6, 1e6)
        ccz = jnp.clip(gz, -1e6, 1e6)
        pos = (snx * ccx + sny * ccy + snz * ccz) / sq3
        for ch, val in enumerate((rho, theta, phi, ux, uy, uz, pos,
                                  gx, gy, gz)):
            feat_ref[ch, 0, i] = val


def _geom_call(xg, carr):
    return pl.pallas_call(
        _geom_body,
        grid=(B,),
        in_specs=[
            pl.BlockSpec((3, NG, 1, 32, 128), lambda b: (0, 0, b, 0, 0)),
            pl.BlockSpec((1, 3, 32, 128), lambda b: (b, 0, 0, 0)),
        ],
        out_specs=pl.BlockSpec((C, 1, NG, 32, 128), lambda b: (0, b, 0, 0, 0)),
        out_shape=jax.ShapeDtypeStruct((C, B, NG, 32, 128), jnp.float32),
    )(xg, carr)


_M = B * NG * N
_MS = _M // 128


def _mlp_body(f_ref, w1_ref, g1_ref, be1_ref, w2_ref, cb2_ref, g2_ref,
              be2_ref, w3_ref, cb3_ref, out_ref):
    inv_m = jnp.float32(1.0 / _M)

    def matmul(w_ref, src, bias_ref=None):
        out = []
        for d in range(C):
            acc = w_ref[d, 0] * src[0]
            for c in range(1, C):
                acc = acc + w_ref[d, c] * src[c]
            if bias_ref is not None:
                acc = acc + bias_ref[d]
            out.append(acc)
        return out

    def bn_relu(xs, g_ref, be_ref):
        out = []
        for d in range(C):
            x = xs[d]
            s = jnp.sum(jnp.sum(x, axis=1, keepdims=True), axis=0,
                        keepdims=True)
            s2 = jnp.sum(jnp.sum(x * x, axis=1, keepdims=True), axis=0,
                         keepdims=True)
            mean = s * inv_m
            var = s2 * inv_m - mean * mean
            xh = (x - mean) / jnp.sqrt(var + 1e-5)
            out.append(jnp.maximum(xh * g_ref[d] + be_ref[d], 0.0))
        return out

    feats = [f_ref[c] for c in range(C)]
    x = matmul(w1_ref, feats)
    x = bn_relu(x, g1_ref, be1_ref)
    x = matmul(w2_ref, x, cb2_ref)
    x = bn_relu(x, g2_ref, be2_ref)
    x = matmul(w3_ref, x, cb3_ref)
    for d in range(C):
        v = x[d].reshape(B, NG, 32, 128)
        out_ref[d] = jnp.sum(v, axis=1)


def _mlp_call(feat, W1, g1, be1, W2, cb2, g2, be2, W3, cb3):
    smem = pl.BlockSpec(memory_space=pltpu.MemorySpace.SMEM)
    return pl.pallas_call(
        _mlp_body,
        in_specs=[pl.BlockSpec((C, _MS, 128), lambda: (0, 0, 0)),
                  smem, smem, smem, smem, smem, smem, smem, smem, smem],
        out_specs=pl.BlockSpec((C, B, 32, 128), lambda: (0, 0, 0, 0)),
        out_shape=jax.ShapeDtypeStruct((C, B, 32, 128), jnp.float32),
    )(feat, W1, g1, be1, W2, cb2, g2, be2, W3, cb3)


def kernel(center, W1, g1, be1, W2, cb2, g2, be2, W3, cb3):
    center_t = center.transpose(0, 2, 1)
    knn = _knn_call(center, center_t)

    nbr = knn[:, :, 1:KNN]
    gidx3 = (nbr * 3).transpose(0, 2, 1)
    gidx3 = gidx3.reshape(B, NG, N // QPT, QPT)
    gidx3 = gidx3.transpose(0, 2, 1, 3).reshape(NTILES, NG, QPT)
    tab = center.reshape(B, 3 * N)
    gath = _sc_gather_call(tab, gidx3)

    xg = gath.reshape(B, N // QPT, 3, NG, QPT)
    xg = xg.transpose(2, 3, 0, 1, 4).reshape(3, NG, B, 32, 128)
    carr = center_t.reshape(B, 3, 32, 128)
    feat = _geom_call(xg, carr)

    featm = feat.reshape(C, _MS, 128)
    out = _mlp_call(featm, W1, g1, be1, W2, cb2, g2, be2, W3, cb3)
    return out.reshape(C, B, N).transpose(1, 0, 2)


# speedup vs baseline: 1.0478x; 1.0478x over previous
import functools
import math

import jax
import jax.numpy as jnp
from jax import lax
from jax.experimental import pallas as pl
from jax.experimental.pallas import tpu as pltpu
from jax.experimental.pallas import tpu_sc as plsc

B, N, KNN = 4, 4096, 9
NG = 8
C = 10
QB = 512
NTILES = 32
QPT = (B * N) // NTILES
BIGF = 3.0e38


def _knn_body(q_ref, d_ref, out_ref):
    q = q_ref[0]
    d = d_ref[0]
    qx, qy, qz = q[:, 0:1], q[:, 1:2], q[:, 2:3]
    dx, dy, dz = d[0:1, :], d[1:2, :], d[2:3, :]
    bq = lambda v: v.astype(jnp.bfloat16).astype(jnp.float32)
    t = bq(qx) * bq(dx) + bq(qy) * bq(dy) + bq(qz) * bq(dz)
    qn = qx * qx + qy * qy + qz * qz
    dn = dx * dx + dy * dy + dz * dz
    dist = -2.0 * t + qn + dn
    iota = lax.broadcasted_iota(jnp.int32, (QB, N), 1).astype(jnp.float32)
    lane16 = lax.broadcasted_iota(jnp.int32, (QB, 16), 1)
    outv = jnp.zeros((QB, 16), jnp.int32)
    for j in range(KNN):
        m = jnp.min(dist, axis=1, keepdims=True)
        cand = jnp.where(dist == m, iota, jnp.float32(N))
        idx = jnp.min(cand, axis=1, keepdims=True)
        outv = jnp.where(lane16 == j, idx.astype(jnp.int32), outv)
        if j + 1 < KNN:
            dist = jnp.where(iota == idx, BIGF, dist)
    out_ref[0] = outv


def _knn_call(center, center_t):
    return pl.pallas_call(
        _knn_body,
        grid=(B, N // QB),
        in_specs=[
            pl.BlockSpec((1, QB, 3), lambda b, q: (b, q, 0)),
            pl.BlockSpec((1, 3, N), lambda b, q: (b, 0, 0)),
        ],
        out_specs=pl.BlockSpec((1, QB, 16), lambda b, q: (b, q, 0)),
        out_shape=jax.ShapeDtypeStruct((B, N, 16), jnp.int32),
    )(center, center_t)


def _sc_gather_body(tab_hbm, idx_hbm, out_hbm, tab_v, idx_v, out_v):
    wid = lax.axis_index("s") * 2 + lax.axis_index("c")
    b = wid // (N // QPT)
    pltpu.sync_copy(tab_hbm.at[b], tab_v)
    pltpu.sync_copy(idx_hbm.at[wid], idx_v)

    def chunk(i, _):
        for j in range(NG):
            iv = idx_v[j, pl.ds(i * 16, 16)]
            for c in range(3):
                out_v[c, j, pl.ds(i * 16, 16)] = plsc.load_gather(
                    tab_v, [iv + c])
        return _

    lax.fori_loop(0, QPT // 16, chunk, 0)
    pltpu.sync_copy(out_v, out_hbm.at[wid])


def _sc_gather_call(tab, gidx3):
    kfn = functools.partial(
        pl.kernel,
        mesh=plsc.VectorSubcoreMesh(core_axis_name="c", subcore_axis_name="s"),
        compiler_params=pltpu.CompilerParams(needs_layout_passes=False),
        out_type=jax.ShapeDtypeStruct((NTILES, 3, NG, QPT), jnp.float32),
        scratch_types=[
            pltpu.VMEM((3 * N,), jnp.float32),
            pltpu.VMEM((NG, QPT), jnp.int32),
            pltpu.VMEM((3, NG, QPT), jnp.float32),
        ],
    )(_sc_gather_body)
    return kfn(tab, gidx3)


_EPS = 1e-10
_CE_PAIRS = [(0, 1), (2, 3), (4, 5), (6, 7),
             (0, 2), (1, 3), (4, 6), (5, 7),
             (1, 2), (5, 6),
             (0, 4), (1, 5), (2, 6), (3, 7),
             (2, 4), (3, 5),
             (1, 2), (3, 4), (5, 6)]


def _atan2_phi(y, x):
    xa = x + 1e-10 * (jnp.abs(x) < 1e-10).astype(jnp.float32)
    ya = y + 1e-10 * (jnp.abs(y) < 1e-10).astype(jnp.float32)
    return jnp.arctan2(ya, xa) / (2.0 * math.pi) + 0.5


def _geom_body(xg_ref, c_ref, feat_ref):
    cx, cy, cz = c_ref[0, 0], c_ref[0, 1], c_ref[0, 2]
    phis, js, xs, ys, zs = [], [], [], [], []
    for j in range(NG):
        x = xg_ref[0, j, 0] - cx
        y = xg_ref[1, j, 0] - cy
        z = xg_ref[2, j, 0] - cz
        xs.append(x); ys.append(y); zs.append(z)
        phis.append(_atan2_phi(y, x))
        js.append(jnp.full(x.shape, j, jnp.int32))
    for (a, b) in _CE_PAIRS:
        swap = (phis[a] > phis[b]) | ((phis[a] == phis[b]) & (js[a] > js[b]))
        for arr in (phis, js, xs, ys, zs):
            na = jnp.where(swap, arr[b], arr[a])
            nb = jnp.where(swap, arr[a], arr[b])
            arr[a], arr[b] = na, nb

    sq3 = jnp.float32(math.sqrt(3.0) + 1e-6)
    smask = None
    feats = []
    for i in range(NG):
        k = (i + 1) % NG
        sx, sy, sz = xs[i], ys[i], zs[i]
        rx, ry, rz = xs[k], ys[k], zs[k]
        nx = sy * rz - sz * ry
        ny = sz * rx - sx * rz
        nz = sx * ry - sy * rx
        norm = jnp.sqrt(nx * nx + ny * ny + nz * nz)
        safe = jnp.where(norm < 1e-6, jnp.float32(1.0), norm)
        ux, uy, uz = nx / safe, ny / safe, nz / safe
        if i == 0:
            smask = jnp.where(ux > 0, jnp.float32(1.0), jnp.float32(-1.0))
        feats.append((ux, uy, uz, sx, sy, sz, rx, ry, rz))

    for i in range(NG):
        ux, uy, uz, sx, sy, sz, rx, ry, rz = feats[i]
        ux, uy, uz = ux * smask, uy * smask, uz * smask
        gx, gy, gz = (sx + rx) / 3.0, (sy + ry) / 3.0, (sz + rz) / 3.0
        rho = jnp.sqrt(gx * gx + gy * gy + gz * gz + _EPS)
        zdr = gz / jnp.maximum(rho, _EPS)
        zdr = jnp.clip(zdr, -1.0 + _EPS, 1.0 - _EPS)
        theta = jnp.arctan2(jnp.sqrt((1.0 - zdr) * (1.0 + zdr)), zdr) / math.pi
        phi = _atan2_phi(gy, gx)
        nn = jnp.sqrt(ux * ux + uy * uy + uz * uz)
        nsafe = jnp.maximum(nn, 1e-6)
        small = nn < 1e-6
        snx = jnp.where(small, jnp.float32(1.0), ux / nsafe)
        sny = jnp.where(small, jnp.float32(0.0), uy / nsafe)
        snz = jnp.where(small, jnp.float32(0.0), uz / nsafe)
        ccx = jnp.clip(gx, -1e6, 1e6)
        ccy = jnp.clip(gy, -1e6, 1e6)
        ccz = jnp.clip(gz, -1e6, 1e6)
        pos = (snx * ccx + sny * ccy + snz * ccz) / sq3
        for ch, val in enumerate((rho, theta, phi, ux, uy, uz, pos,
                                  gx, gy, gz)):
            feat_ref[ch, 0, i] = val


def _geom_call(xg, carr):
    return pl.pallas_call(
        _geom_body,
        grid=(B,),
        in_specs=[
            pl.BlockSpec((3, NG, 1, 32, 128), lambda b: (0, 0, b, 0, 0)),
            pl.BlockSpec((1, 3, 32, 128), lambda b: (b, 0, 0, 0)),
        ],
        out_specs=pl.BlockSpec((C, 1, NG, 32, 128), lambda b: (0, b, 0, 0, 0)),
        out_shape=jax.ShapeDtypeStruct((C, B, NG, 32, 128), jnp.float32),
    )(xg, carr)


_M = B * NG * N
_MS = _M // 128


def _mlp_body(f_ref, w1_ref, g1_ref, be1_ref, w2_ref, cb2_ref, g2_ref,
              be2_ref, w3_ref, cb3_ref, out_ref):
    inv_m = jnp.float32(1.0 / _M)

    def matmul(w_ref, src, bias_ref=None):
        out = []
        for d in range(C):
            acc = w_ref[d, 0] * src[0]
            for c in range(1, C):
                acc = acc + w_ref[d, c] * src[c]
            if bias_ref is not None:
                acc = acc + bias_ref[d]
            out.append(acc)
        return out

    def bn_relu(xs, g_ref, be_ref):
        out = []
        for d in range(C):
            x = xs[d]
            s = jnp.sum(jnp.sum(x, axis=1, keepdims=True), axis=0,
                        keepdims=True)
            s2 = jnp.sum(jnp.sum(x * x, axis=1, keepdims=True), axis=0,
                         keepdims=True)
            mean = s * inv_m
            var = s2 * inv_m - mean * mean
            xh = (x - mean) / jnp.sqrt(var + 1e-5)
            out.append(jnp.maximum(xh * g_ref[d] + be_ref[d], 0.0))
        return out

    feats = [f_ref[c] for c in range(C)]
    x = matmul(w1_ref, feats)
    x = bn_relu(x, g1_ref, be1_ref)
    x = matmul(w2_ref, x, cb2_ref)
    x = bn_relu(x, g2_ref, be2_ref)
    x = matmul(w3_ref, x, cb3_ref)
    for d in range(C):
        v = x[d].reshape(B, NG, 32, 128)
        out_ref[d] = jnp.sum(v, axis=1)


def _mlp_call(feat, W1, g1, be1, W2, cb2, g2, be2, W3, cb3):
    smem = pl.BlockSpec(memory_space=pltpu.MemorySpace.SMEM)
    return pl.pallas_call(
        _mlp_body,
        in_specs=[pl.BlockSpec((C, _MS, 128), lambda: (0, 0, 0)),
                  smem, smem, smem, smem, smem, smem, smem, smem, smem],
        out_specs=pl.BlockSpec((C, B, 32, 128), lambda: (0, 0, 0, 0)),
        out_shape=jax.ShapeDtypeStruct((C, B, 32, 128), jnp.float32),
    )(feat, W1, g1, be1, W2, cb2, g2, be2, W3, cb3)


def kernel(center, W1, g1, be1, W2, cb2, g2, be2, W3, cb3):
    center_t = center.transpose(0, 2, 1)
    knn = _knn_call(center, center_t)

    nbr = knn[:, :, 1:KNN]
    gidx3 = (nbr * 3).transpose(0, 2, 1)
    gidx3 = gidx3.reshape(B, NG, N // QPT, QPT)
    gidx3 = gidx3.transpose(0, 2, 1, 3).reshape(NTILES, NG, QPT)
    tab = center.reshape(B, 3 * N)
    gath = _sc_gather_call(tab, gidx3)

    xg = gath.reshape(B, N // QPT, 3, NG, QPT)
    xg = xg.transpose(2, 3, 0, 1, 4).reshape(3, NG, B, 32, 128)
    carr = center_t.reshape(B, 3, 32, 128)
    feat = _geom_call(xg, carr)

    featm = feat.reshape(C, _MS, 128)
    out = _mlp_call(featm, W1, g1, be1, W2, cb2, g2, be2, W3, cb3)
    return out.reshape(C, B, N).transpose(1, 0, 2)


# speedup vs baseline: 1.1231x; 1.0718x over previous
import functools
import math

import jax
import jax.numpy as jnp
from jax import lax
from jax.experimental import pallas as pl
from jax.experimental.pallas import tpu as pltpu
from jax.experimental.pallas import tpu_sc as plsc

B, N, KNN = 4, 4096, 9
NG = 8
C = 10
QB = 512
NTILES = 32
QPT = (B * N) // NTILES
BIGF = 3.0e38


def _knn_body(q_ref, d_ref, out_ref):
    q = q_ref[0]
    d = d_ref[0]
    qx, qy, qz = q[:, 0:1], q[:, 1:2], q[:, 2:3]
    dx, dy, dz = d[0:1, :], d[1:2, :], d[2:3, :]
    t = jax.lax.dot_general(q.astype(jnp.bfloat16), d.astype(jnp.bfloat16),
                            (((1,), (0,)), ((), ())),
                            preferred_element_type=jnp.float32)
    qn = qx * qx + qy * qy + qz * qz
    dn = dx * dx + dy * dy + dz * dz
    dist = -2.0 * t + qn + dn
    iota = lax.broadcasted_iota(jnp.int32, (QB, N), 1).astype(jnp.float32)
    lane16 = lax.broadcasted_iota(jnp.int32, (QB, 16), 1)
    outv = jnp.zeros((QB, 16), jnp.int32)
    for j in range(KNN):
        m = jnp.min(dist, axis=1, keepdims=True)
        cand = jnp.where(dist == m, iota, jnp.float32(N))
        idx = jnp.min(cand, axis=1, keepdims=True)
        outv = jnp.where(lane16 == j, idx.astype(jnp.int32), outv)
        if j + 1 < KNN:
            dist = jnp.where(iota == idx, BIGF, dist)
    out_ref[0] = outv


def _knn_call(center, center_t):
    return pl.pallas_call(
        _knn_body,
        grid=(B, N // QB),
        in_specs=[
            pl.BlockSpec((1, QB, 3), lambda b, q: (b, q, 0)),
            pl.BlockSpec((1, 3, N), lambda b, q: (b, 0, 0)),
        ],
        out_specs=pl.BlockSpec((1, QB, 16), lambda b, q: (b, q, 0)),
        out_shape=jax.ShapeDtypeStruct((B, N, 16), jnp.int32),
    )(center, center_t)


def _sc_gather_body(tab_hbm, idx_hbm, out_hbm, tab_v, idx_v, out_v):
    wid = lax.axis_index("s") * 2 + lax.axis_index("c")
    b = wid // (N // QPT)
    pltpu.sync_copy(tab_hbm.at[b], tab_v)
    pltpu.sync_copy(idx_hbm.at[wid], idx_v)

    def chunk(i, _):
        for j in range(NG):
            iv = idx_v[j, pl.ds(i * 16, 16)]
            for c in range(3):
                out_v[c, j, pl.ds(i * 16, 16)] = plsc.load_gather(
                    tab_v, [iv + c])
        return _

    lax.fori_loop(0, QPT // 16, chunk, 0)
    pltpu.sync_copy(out_v, out_hbm.at[wid])


def _sc_gather_call(tab, gidx3):
    kfn = functools.partial(
        pl.kernel,
        mesh=plsc.VectorSubcoreMesh(core_axis_name="c", subcore_axis_name="s"),
        compiler_params=pltpu.CompilerParams(needs_layout_passes=False),
        out_type=jax.ShapeDtypeStruct((NTILES, 3, NG, QPT), jnp.float32),
        scratch_types=[
            pltpu.VMEM((3 * N,), jnp.float32),
            pltpu.VMEM((NG, QPT), jnp.int32),
            pltpu.VMEM((3, NG, QPT), jnp.float32),
        ],
    )(_sc_gather_body)
    return kfn(tab, gidx3)


_EPS = 1e-10
_CE_PAIRS = [(0, 1), (2, 3), (4, 5), (6, 7),
             (0, 2), (1, 3), (4, 6), (5, 7),
             (1, 2), (5, 6),
             (0, 4), (1, 5), (2, 6), (3, 7),
             (2, 4), (3, 5),
             (1, 2), (3, 4), (5, 6)]


def _atan2_phi(y, x):
    xa = x + 1e-10 * (jnp.abs(x) < 1e-10).astype(jnp.float32)
    ya = y + 1e-10 * (jnp.abs(y) < 1e-10).astype(jnp.float32)
    return jnp.arctan2(ya, xa) / (2.0 * math.pi) + 0.5


def _geom_body(xg_ref, c_ref, feat_ref):
    cx, cy, cz = c_ref[0, 0], c_ref[0, 1], c_ref[0, 2]
    phis, js, xs, ys, zs = [], [], [], [], []
    for j in range(NG):
        x = xg_ref[0, j, 0] - cx
        y = xg_ref[1, j, 0] - cy
        z = xg_ref[2, j, 0] - cz
        xs.append(x); ys.append(y); zs.append(z)
        phis.append(_atan2_phi(y, x))
        js.append(jnp.full(x.shape, j, jnp.int32))
    for (a, b) in _CE_PAIRS:
        swap = (phis[a] > phis[b]) | ((phis[a] == phis[b]) & (js[a] > js[b]))
        for arr in (phis, js, xs, ys, zs):
            na = jnp.where(swap, arr[b], arr[a])
            nb = jnp.where(swap, arr[a], arr[b])
            arr[a], arr[b] = na, nb

    sq3 = jnp.float32(math.sqrt(3.0) + 1e-6)
    smask = None
    feats = []
    for i in range(NG):
        k = (i + 1) % NG
        sx, sy, sz = xs[i], ys[i], zs[i]
        rx, ry, rz = xs[k], ys[k], zs[k]
        nx = sy * rz - sz * ry
        ny = sz * rx - sx * rz
        nz = sx * ry - sy * rx
        norm = jnp.sqrt(nx * nx + ny * ny + nz * nz)
        safe = jnp.where(norm < 1e-6, jnp.float32(1.0), norm)
        ux, uy, uz = nx / safe, ny / safe, nz / safe
        if i == 0:
            smask = jnp.where(ux > 0, jnp.float32(1.0), jnp.float32(-1.0))
        feats.append((ux, uy, uz, sx, sy, sz, rx, ry, rz))

    for i in range(NG):
        ux, uy, uz, sx, sy, sz, rx, ry, rz = feats[i]
        ux, uy, uz = ux * smask, uy * smask, uz * smask
        gx, gy, gz = (sx + rx) / 3.0, (sy + ry) / 3.0, (sz + rz) / 3.0
        rho = jnp.sqrt(gx * gx + gy * gy + gz * gz + _EPS)
        zdr = gz / jnp.maximum(rho, _EPS)
        zdr = jnp.clip(zdr, -1.0 + _EPS, 1.0 - _EPS)
        theta = jnp.arctan2(jnp.sqrt((1.0 - zdr) * (1.0 + zdr)), zdr) / math.pi
        phi = _atan2_phi(gy, gx)
        nn = jnp.sqrt(ux * ux + uy * uy + uz * uz)
        nsafe = jnp.maximum(nn, 1e-6)
        small = nn < 1e-6
        snx = jnp.where(small, jnp.float32(1.0), ux / nsafe)
        sny = jnp.where(small, jnp.float32(0.0), uy / nsafe)
        snz = jnp.where(small, jnp.float32(0.0), uz / nsafe)
        ccx = jnp.clip(gx, -1e6, 1e6)
        ccy = jnp.clip(gy, -1e6, 1e6)
        ccz = jnp.clip(gz, -1e6, 1e6)
        pos = (snx * ccx + sny * ccy + snz * ccz) / sq3
        for ch, val in enumerate((rho, theta, phi, ux, uy, uz, pos,
                                  gx, gy, gz)):
            feat_ref[ch, 0, i] = val


def _geom_call(xg, carr):
    return pl.pallas_call(
        _geom_body,
        grid=(B,),
        in_specs=[
            pl.BlockSpec((3, NG, 1, 32, 128), lambda b: (0, 0, b, 0, 0)),
            pl.BlockSpec((1, 3, 32, 128), lambda b: (b, 0, 0, 0)),
        ],
        out_specs=pl.BlockSpec((C, 1, NG, 32, 128), lambda b: (0, b, 0, 0, 0)),
        out_shape=jax.ShapeDtypeStruct((C, B, NG, 32, 128), jnp.float32),
    )(xg, carr)


_M = B * NG * N
_MS = _M // 128


def _mlp_body(f_ref, w1_ref, g1_ref, be1_ref, w2_ref, cb2_ref, g2_ref,
              be2_ref, w3_ref, cb3_ref, out_ref):
    inv_m = jnp.float32(1.0 / _M)

    def matmul(w_ref, src, bias_ref=None):
        out = []
        for d in range(C):
            acc = w_ref[d, 0] * src[0]
            for c in range(1, C):
                acc = acc + w_ref[d, c] * src[c]
            if bias_ref is not None:
                acc = acc + bias_ref[d]
            out.append(acc)
        return out

    def bn_relu(xs, g_ref, be_ref):
        out = []
        for d in range(C):
            x = xs[d]
            s = jnp.sum(jnp.sum(x, axis=1, keepdims=True), axis=0,
                        keepdims=True)
            s2 = jnp.sum(jnp.sum(x * x, axis=1, keepdims=True), axis=0,
                         keepdims=True)
            mean = s * inv_m
            var = s2 * inv_m - mean * mean
            xh = (x - mean) / jnp.sqrt(var + 1e-5)
            out.append(jnp.maximum(xh * g_ref[d] + be_ref[d], 0.0))
        return out

    feats = [f_ref[c] for c in range(C)]
    x = matmul(w1_ref, feats)
    x = bn_relu(x, g1_ref, be1_ref)
    x = matmul(w2_ref, x, cb2_ref)
    x = bn_relu(x, g2_ref, be2_ref)
    x = matmul(w3_ref, x, cb3_ref)
    for d in range(C):
        v = x[d].reshape(B, NG, 32, 128)
        out_ref[d] = jnp.sum(v, axis=1)


def _mlp_call(feat, W1, g1, be1, W2, cb2, g2, be2, W3, cb3):
    smem = pl.BlockSpec(memory_space=pltpu.MemorySpace.SMEM)
    return pl.pallas_call(
        _mlp_body,
        in_specs=[pl.BlockSpec((C, _MS, 128), lambda: (0, 0, 0)),
                  smem, smem, smem, smem, smem, smem, smem, smem, smem],
        out_specs=pl.BlockSpec((C, B, 32, 128), lambda: (0, 0, 0, 0)),
        out_shape=jax.ShapeDtypeStruct((C, B, 32, 128), jnp.float32),
    )(feat, W1, g1, be1, W2, cb2, g2, be2, W3, cb3)


def kernel(center, W1, g1, be1, W2, cb2, g2, be2, W3, cb3):
    center_t = center.transpose(0, 2, 1)
    knn = _knn_call(center, center_t)

    nbr = knn[:, :, 1:KNN]
    gidx3 = (nbr * 3).transpose(0, 2, 1)
    gidx3 = gidx3.reshape(B, NG, N // QPT, QPT)
    gidx3 = gidx3.transpose(0, 2, 1, 3).reshape(NTILES, NG, QPT)
    tab = center.reshape(B, 3 * N)
    gath = _sc_gather_call(tab, gidx3)

    xg = gath.reshape(B, N // QPT, 3, NG, QPT)
    xg = xg.transpose(2, 3, 0, 1, 4).reshape(3, NG, B, 32, 128)
    carr = center_t.reshape(B, 3, 32, 128)
    feat = _geom_call(xg, carr)

    featm = feat.reshape(C, _MS, 128)
    out = _mlp_call(featm, W1, g1, be1, W2, cb2, g2, be2, W3, cb3)
    return out.reshape(C, B, N).transpose(1, 0, 2)
